# baseline (speedup 1.0000x reference)
import functools
import math

import jax
import jax.numpy as jnp
from jax import lax
from jax.experimental import pallas as pl
from jax.experimental.pallas import tpu as pltpu
from jax.experimental.pallas import tpu_sc as plsc

N = 50000
E = 800000
D = 64
L = 3

NW = 32
EW_REAL = 25000
EW = 25600
E_PAD = NW * EW
CH = 512
NCH = EW // CH
RING = 3
NP = 51200
TRASH = NP - 1
RPT = NP // 16
PB = 2000
EB = 8192
PR = CH // 256

_mesh = plsc.VectorSubcoreMesh(core_axis_name="c", subcore_axis_name="s")
_sc_params = pltpu.CompilerParams(use_tc_tiling_on_sc=False)


def _prep_body(x_ref, w_ref, e3_ref, q3r, kr, vr, q3c, kc, vc):
    x = x_ref[...]
    mm = lambda w: lax.dot_general(x, w, (((1,), (0,)), ((), ())),
                                   preferred_element_type=jnp.float32)
    q3r[...] = mm(w_ref[0])[None, :, :] * e3_ref[0][:, None, :]
    kr[...] = mm(w_ref[1])
    vr[...] = mm(w_ref[2])
    q3c[...] = mm(w_ref[3])[None, :, :] * e3_ref[1][:, None, :]
    kc[...] = mm(w_ref[4])
    vc[...] = mm(w_ref[5])


def _prep(x, ws, e3s):
    return pl.pallas_call(
        _prep_body,
        grid=(N // PB,),
        in_specs=[
            pl.BlockSpec((PB, D), lambda i: (i, 0)),
            pl.BlockSpec((6, D, D), lambda i: (0, 0, 0)),
            pl.BlockSpec((2, 3, D), lambda i: (0, 0, 0)),
        ],
        out_specs=[
            pl.BlockSpec((3, PB, D), lambda i: (0, i, 0)),
            pl.BlockSpec((PB, D), lambda i: (i, 0)),
            pl.BlockSpec((PB, D), lambda i: (i, 0)),
            pl.BlockSpec((3, PB, D), lambda i: (0, i, 0)),
            pl.BlockSpec((PB, D), lambda i: (i, 0)),
            pl.BlockSpec((PB, D), lambda i: (i, 0)),
        ],
        out_shape=[
            jax.ShapeDtypeStruct((3, N, D), jnp.float32),
            jax.ShapeDtypeStruct((N, D), jnp.float32),
            jax.ShapeDtypeStruct((N, D), jnp.float32),
            jax.ShapeDtypeStruct((3, N, D), jnp.float32),
            jax.ShapeDtypeStruct((N, D), jnp.float32),
            jax.ShapeDtypeStruct((N, D), jnp.float32),
        ],
    )(x, ws, e3s)


@functools.partial(
    pl.kernel,
    out_type=[jax.ShapeDtypeStruct((E_PAD, D), jnp.float32)] * 6,
    mesh=_mesh,
    compiler_params=_sc_params,
    scratch_types=[
        pltpu.VMEM((EW,), jnp.int32),
        pltpu.VMEM((CH, D), jnp.float32), pltpu.VMEM((CH, D), jnp.float32),
        pltpu.VMEM((CH, D), jnp.float32),
        pltpu.SemaphoreType.DMA, pltpu.SemaphoreType.DMA,
        pltpu.SemaphoreType.DMA, pltpu.SemaphoreType.DMA,
        pltpu.SemaphoreType.DMA, pltpu.SemaphoreType.DMA,
    ],
)
def _gather6(q3r, kr, vr, q3c, kc, vc, idxq_r, idxkv_r, idxq_c, idxkv_c,
             qgr, kgr, vgr, qgc, kgc, vgc,
             idx_res, s0, s1, s2, g0, g1, g2, w0, w1, w2):
    wid = lax.axis_index("s") * 2 + lax.axis_index("c")
    base = wid * EW
    sb = (s0, s1, s2)
    gs = (g0, g1, g2)
    wsm = (w0, w1, w2)

    def phase(table, idx_row, out):
        pltpu.sync_copy(idx_row, idx_res)
        for jj in range(RING):
            pltpu.async_copy(table.at[idx_res.at[pl.ds(jj * CH, CH)]],
                             sb[jj], gs[jj])

        def body(j, c):
            b = lax.rem(j, RING)

            def slot(b_):
                pltpu.make_async_copy(
                    table.at[idx_res.at[pl.ds(0, CH)]], sb[b_], gs[b_]).wait()
                pltpu.async_copy(sb[b_], out.at[pl.ds(base + j * CH, CH)],
                                 wsm[b_])

                @pl.when(j < NCH - RING)
                def _():
                    pltpu.make_async_copy(
                        sb[b_], out.at[pl.ds(base, CH)], wsm[b_]).wait()
                    pltpu.async_copy(
                        table.at[idx_res.at[pl.ds((j + RING) * CH, CH)]],
                        sb[b_], gs[b_])

            for b_ in range(RING):
                pl.when(b == b_)(functools.partial(slot, b_))
            return c

        lax.fori_loop(0, NCH, body, 0, unroll=False)
        for b_ in range(RING):
            pltpu.make_async_copy(sb[b_], out.at[pl.ds(base, CH)],
                                  wsm[b_]).wait()

    phase(q3r, idxq_r.at[wid], qgr)
    phase(kr, idxkv_r.at[wid], kgr)
    phase(vr, idxkv_r.at[wid], vgr)
    phase(q3c, idxq_c.at[wid], qgc)
    phase(kc, idxkv_c.at[wid], kgc)
    phase(vc, idxkv_c.at[wid], vgc)


def _edge_body(qg_ref, kg_ref, vg_ref, lo_ref, hi_ref, p_ref):
    s = jnp.sum(qg_ref[...] * kg_ref[...], axis=1)
    p = jnp.exp(s)
    pv = vg_ref[...] * p[:, None]
    lo_ref[...] = pv[:, : D // 2]
    hi_ref[...] = pv[:, D // 2:]
    p_ref[...] = p.reshape(EB // 256, 256)


def _edge_math(qg, kg, vg):
    return pl.pallas_call(
        _edge_body,
        grid=(E_PAD // EB,),
        in_specs=[pl.BlockSpec((EB, D), lambda i: (i, 0))] * 3,
        out_specs=[
            pl.BlockSpec((EB, D // 2), lambda i: (i, 0)),
            pl.BlockSpec((EB, D // 2), lambda i: (i, 0)),
            pl.BlockSpec((EB // 256, 256), lambda i: (i, 0)),
        ],
        out_shape=[
            jax.ShapeDtypeStruct((E_PAD, D // 2), jnp.float32),
            jax.ShapeDtypeStruct((E_PAD, D // 2), jnp.float32),
            jax.ShapeDtypeStruct((E_PAD // 256, 256), jnp.float32),
        ],
    )(qg, kg, vg)


@functools.partial(
    pl.kernel,
    out_type=[jax.ShapeDtypeStruct((2, 2, 4, NP, D // 4), jnp.float32),
              jax.ShapeDtypeStruct((2, 2, NP), jnp.float32)],
    mesh=_mesh,
    compiler_params=_sc_params,
    scratch_types=[
        pltpu.VMEM((EW,), jnp.int32),
        pltpu.VMEM((EW,), jnp.float32),
        pltpu.VMEM((CH, D // 4), jnp.float32),
        pltpu.VMEM((CH, D // 4), jnp.float32),
        pltpu.VMEM((CH, D // 4), jnp.float32),
        pltpu.VMEM_SHARED((NP, D // 4), jnp.float32),
        pltpu.VMEM_SHARED((NP,), jnp.float32),
        pltpu.SemaphoreType.DMA, pltpu.SemaphoreType.DMA,
        pltpu.SemaphoreType.DMA, pltpu.SemaphoreType.DMA,
        pltpu.SemaphoreType.DMA, pltpu.SemaphoreType.DMA,
        pltpu.SemaphoreType.DMA,
    ],
)
def _scatter2(lo_r, hi_r, p_r, sidx_r, lo_c, hi_c, p_c, sidx_c, z2, z1,
              accs, dens,
              sidx_res, p_res, v0, v1, v2, acc, den,
              l0, l1, l2, a0, a1, a2, dsem):
    cc = lax.axis_index("c")
    t = lax.axis_index("s")
    wid = t * 2 + cc
    vb = (v0, v1, v2)
    lsm = (l0, l1, l2)
    asx = (a0, a1, a2)
    rows = pl.ds(t * RPT, RPT)

    QW = D // 4

    def agg(srcarr, qc, with_den):
        for jj in range(RING):
            pltpu.async_copy(
                srcarr.at[pl.ds(wid * EW + jj * CH, CH), pl.ds(qc * QW, QW)],
                vb[jj], lsm[jj])

        def body(j, c):
            b = lax.rem(j, RING)
            sl = sidx_res.at[pl.ds(j * CH, CH)]

            def slot(b_):
                pltpu.make_async_copy(
                    srcarr.at[pl.ds(wid * EW, CH), pl.ds(0, QW)],
                    vb[b_], lsm[b_]).wait()
                pltpu.async_copy(vb[b_], acc.at[sl], asx[b_], add=True)
                if with_den:
                    pltpu.async_copy(p_res.at[pl.ds(j * CH, CH)], den.at[sl],
                                     dsem, add=True)

                @pl.when(j < NCH - RING)
                def _():
                    pltpu.make_async_copy(vb[b_], acc.at[sl], asx[b_]).wait()
                    pltpu.async_copy(
                        srcarr.at[pl.ds(wid * EW + (j + RING) * CH, CH),
                                  pl.ds(qc * QW, QW)],
                        vb[b_], lsm[b_])

            for b_ in range(RING):
                pl.when(b == b_)(functools.partial(slot, b_))
            return c

        lax.fori_loop(0, NCH, body, 0, unroll=False)
        for b_ in range(RING):
            pltpu.make_async_copy(vb[b_], acc.at[sidx_res.at[pl.ds(0, CH)]],
                                  asx[b_]).wait()
        if with_den:
            for _ in range(NCH):
                pltpu.make_async_copy(p_res.at[pl.ds(0, CH)],
                                      den.at[sidx_res.at[pl.ds(0, CH)]],
                                      dsem).wait()

    pltpu.sync_copy(z2, acc.at[rows])
    pltpu.sync_copy(z1, den.at[rows])
    plsc.subcore_barrier()

    for d, (lo, hi, p2, sidx) in enumerate(
            ((lo_r, hi_r, p_r, sidx_r), (lo_c, hi_c, p_c, sidx_c))):
        pltpu.sync_copy(sidx.at[wid], sidx_res)
        pltpu.sync_copy(p2.at[wid], p_res)
        for qi, (half, qc) in enumerate(((lo, 0), (lo, 1), (hi, 0), (hi, 1))):
            first = qi == 0
            agg(half, qc, first)
            plsc.subcore_barrier()
            pltpu.sync_copy(acc.at[rows], accs.at[d, cc, qi, rows])
            if first:
                pltpu.sync_copy(den.at[rows], dens.at[d, cc, rows])
                pltpu.sync_copy(z1, den.at[rows])
            pltpu.sync_copy(z2, acc.at[rows])
            plsc.subcore_barrier()


def _pad_idx(a, fill):
    ap = jnp.full((NW, EW), fill, jnp.int32)
    return ap.at[:, :EW_REAL].set(a.reshape(NW, EW_REAL).astype(jnp.int32))


def kernel(params, atoms, edge_index, edge_ids):
    x = params["atom_table"][atoms]
    src, dst = edge_index[0], edge_index[1]
    eid = edge_ids
    inv = 1.0 / math.sqrt(D)

    idxq_r = _pad_idx(eid * N + dst, 0)
    idxq_c = _pad_idx(eid * N + src, 0)
    idxkv_r = _pad_idx(src, 0)
    idxkv_c = _pad_idx(dst, 0)
    sidx_r = _pad_idx(dst, TRASH)
    sidx_c = _pad_idx(src, TRASH)
    z2 = jnp.zeros((RPT, D // 4), jnp.float32)
    z1 = jnp.zeros((RPT,), jnp.float32)

    for l in range(L):
        ws = jnp.stack([
            params["r2c_Wq"][l], params["r2c_Wk"][l], params["r2c_Wv"][l],
            params["c2r_Wq"][l], params["c2r_Wk"][l], params["c2r_Wv"][l],
        ])
        e3s = jnp.stack([
            jnp.tanh(params["edge_table"] @ params["r2c_We"][l]
                     + params["r2c_be"][l]) * inv,
            jnp.tanh(params["edge_table"] @ params["c2r_We"][l]
                     + params["c2r_be"][l]) * inv,
        ])
        q3r, kr, vr, q3c, kc, vc = _prep(x, ws, e3s)
        qgr, kgr, vgr, qgc, kgc, vgc = _gather6(
            q3r.reshape(3 * N, D), kr, vr, q3c.reshape(3 * N, D), kc, vc,
            idxq_r, idxkv_r, idxq_c, idxkv_c)
        lo_r, hi_r, p_r = _edge_math(qgr, kgr, vgr)
        lo_c, hi_c, p_c = _edge_math(qgc, kgc, vgc)
        accs, dens = _scatter2(lo_r, hi_r, p_r.reshape(NW, EW), sidx_r,
                               lo_c, hi_c, p_c.reshape(NW, EW), sidx_c,
                               z2, z1)
        outs = []
        for d in range(2):
            num = jnp.concatenate(
                [accs[d, 0, q] + accs[d, 1, q] for q in range(4)],
                axis=1)[:N]
            den = (dens[d, 0] + dens[d, 1])[:N]
            outs.append(num / (den[:, None] + 1e-16))
        h = jnp.concatenate(outs, axis=-1) @ params["ffn_W"][l]
        y = h + x
        mu = jnp.mean(y, axis=-1, keepdims=True)
        var = jnp.var(y, axis=-1, keepdims=True)
        x = (y - mu) / jnp.sqrt(var + 1e-5) * params["ln_g"][l] + params["ln_b"][l]
    return x


# speedup vs baseline: 1.0006x; 1.0006x over previous
import functools
import math

import jax
import jax.numpy as jnp
from jax import lax
from jax.experimental import pallas as pl
from jax.experimental.pallas import tpu as pltpu
from jax.experimental.pallas import tpu_sc as plsc

N = 50000
E = 800000
D = 64
L = 3

NW = 32
EW_REAL = 25000
EW = 25600
E_PAD = NW * EW
CH = 512
NCH = EW // CH
RING = 3
NP = 51200
TRASH = NP - 1
RPT = NP // 16
PB = 2000
EB = 8192
PR = CH // 256

_mesh = plsc.VectorSubcoreMesh(core_axis_name="c", subcore_axis_name="s")
_sc_params = pltpu.CompilerParams(use_tc_tiling_on_sc=False)


def _prep_body(x_ref, w_ref, e3_ref, q3r, kr, vr, q3c, kc, vc):
    x = x_ref[...]
    mm = lambda w: lax.dot_general(x, w, (((1,), (0,)), ((), ())),
                                   preferred_element_type=jnp.float32)
    q3r[...] = mm(w_ref[0])[None, :, :] * e3_ref[0][:, None, :]
    kr[...] = mm(w_ref[1])
    vr[...] = mm(w_ref[2])
    q3c[...] = mm(w_ref[3])[None, :, :] * e3_ref[1][:, None, :]
    kc[...] = mm(w_ref[4])
    vc[...] = mm(w_ref[5])


def _prep(x, ws, e3s):
    return pl.pallas_call(
        _prep_body,
        grid=(N // PB,),
        in_specs=[
            pl.BlockSpec((PB, D), lambda i: (i, 0)),
            pl.BlockSpec((6, D, D), lambda i: (0, 0, 0)),
            pl.BlockSpec((2, 3, D), lambda i: (0, 0, 0)),
        ],
        out_specs=[
            pl.BlockSpec((3, PB, D), lambda i: (0, i, 0)),
            pl.BlockSpec((PB, D), lambda i: (i, 0)),
            pl.BlockSpec((PB, D), lambda i: (i, 0)),
            pl.BlockSpec((3, PB, D), lambda i: (0, i, 0)),
            pl.BlockSpec((PB, D), lambda i: (i, 0)),
            pl.BlockSpec((PB, D), lambda i: (i, 0)),
        ],
        out_shape=[
            jax.ShapeDtypeStruct((3, N, D), jnp.float32),
            jax.ShapeDtypeStruct((N, D), jnp.float32),
            jax.ShapeDtypeStruct((N, D), jnp.float32),
            jax.ShapeDtypeStruct((3, N, D), jnp.float32),
            jax.ShapeDtypeStruct((N, D), jnp.float32),
            jax.ShapeDtypeStruct((N, D), jnp.float32),
        ],
    )(x, ws, e3s)


@functools.partial(
    pl.kernel,
    out_type=[jax.ShapeDtypeStruct((E_PAD, D), jnp.float32)] * 6,
    mesh=_mesh,
    compiler_params=_sc_params,
    scratch_types=[
        pltpu.VMEM((EW,), jnp.int32),
        pltpu.VMEM((CH, D), jnp.float32), pltpu.VMEM((CH, D), jnp.float32),
        pltpu.VMEM((CH, D), jnp.float32),
        pltpu.SemaphoreType.DMA, pltpu.SemaphoreType.DMA,
        pltpu.SemaphoreType.DMA, pltpu.SemaphoreType.DMA,
        pltpu.SemaphoreType.DMA, pltpu.SemaphoreType.DMA,
    ],
)
def _gather6(q3r, kr, vr, q3c, kc, vc, idxq_r, idxkv_r, idxq_c, idxkv_c,
             qgr, kgr, vgr, qgc, kgc, vgc,
             idx_res, s0, s1, s2, g0, g1, g2, w0, w1, w2):
    wid = lax.axis_index("s") * 2 + lax.axis_index("c")
    base = wid * EW
    sb = (s0, s1, s2)
    gs = (g0, g1, g2)
    wsm = (w0, w1, w2)

    def phase(table, idx_row, out):
        pltpu.sync_copy(idx_row, idx_res)
        for jj in range(RING - 1):
            pltpu.async_copy(table.at[idx_res.at[pl.ds(jj * CH, CH)]],
                             sb[jj], gs[jj])

        def step(j, first):
            def slot(b_):
                pltpu.make_async_copy(
                    table.at[idx_res.at[pl.ds(0, CH)]], sb[b_], gs[b_]).wait()
                pltpu.async_copy(sb[b_], out.at[pl.ds(base + j * CH, CH)],
                                 wsm[b_])

            def refill(b_):
                if not first:
                    pltpu.make_async_copy(
                        sb[b_], out.at[pl.ds(base, CH)], wsm[b_]).wait()
                pltpu.async_copy(
                    table.at[idx_res.at[pl.ds((j + RING - 1) * CH, CH)]],
                    sb[b_], gs[b_])

            bp = lax.rem(j + RING - 1, RING)

            @pl.when(j + RING - 1 < NCH)
            def _():
                for b_ in range(RING):
                    pl.when(bp == b_)(functools.partial(refill, b_))

            b = lax.rem(j, RING)
            for b_ in range(RING):
                pl.when(b == b_)(functools.partial(slot, b_))

        step(0, True)
        lax.fori_loop(1, NCH, lambda j, c: (step(j, False), c)[1], 0,
                      unroll=False)
        for b_ in range(RING):
            pltpu.make_async_copy(sb[b_], out.at[pl.ds(base, CH)],
                                  wsm[b_]).wait()

    phase(q3r, idxq_r.at[wid], qgr)
    phase(kr, idxkv_r.at[wid], kgr)
    phase(vr, idxkv_r.at[wid], vgr)
    phase(q3c, idxq_c.at[wid], qgc)
    phase(kc, idxkv_c.at[wid], kgc)
    phase(vc, idxkv_c.at[wid], vgc)


def _edge_body(qg_ref, kg_ref, vg_ref, lo_ref, hi_ref, p_ref):
    s = jnp.sum(qg_ref[...] * kg_ref[...], axis=1)
    p = jnp.exp(s)
    pv = vg_ref[...] * p[:, None]
    lo_ref[...] = pv[:, : D // 2]
    hi_ref[...] = pv[:, D // 2:]
    p_ref[...] = p.reshape(EB // 256, 256)


def _edge_math(qg, kg, vg):
    return pl.pallas_call(
        _edge_body,
        grid=(E_PAD // EB,),
        in_specs=[pl.BlockSpec((EB, D), lambda i: (i, 0))] * 3,
        out_specs=[
            pl.BlockSpec((EB, D // 2), lambda i: (i, 0)),
            pl.BlockSpec((EB, D // 2), lambda i: (i, 0)),
            pl.BlockSpec((EB // 256, 256), lambda i: (i, 0)),
        ],
        out_shape=[
            jax.ShapeDtypeStruct((E_PAD, D // 2), jnp.float32),
            jax.ShapeDtypeStruct((E_PAD, D // 2), jnp.float32),
            jax.ShapeDtypeStruct((E_PAD // 256, 256), jnp.float32),
        ],
    )(qg, kg, vg)


@functools.partial(
    pl.kernel,
    out_type=[jax.ShapeDtypeStruct((2, 2, 4, NP, D // 4), jnp.float32),
              jax.ShapeDtypeStruct((2, 2, NP), jnp.float32)],
    mesh=_mesh,
    compiler_params=_sc_params,
    scratch_types=[
        pltpu.VMEM((EW,), jnp.int32),
        pltpu.VMEM((EW,), jnp.float32),
        pltpu.VMEM((CH, D // 4), jnp.float32),
        pltpu.VMEM((CH, D // 4), jnp.float32),
        pltpu.VMEM((CH, D // 4), jnp.float32),
        pltpu.VMEM_SHARED((NP, D // 4), jnp.float32),
        pltpu.VMEM_SHARED((NP,), jnp.float32),
        pltpu.SemaphoreType.DMA, pltpu.SemaphoreType.DMA,
        pltpu.SemaphoreType.DMA, pltpu.SemaphoreType.DMA,
        pltpu.SemaphoreType.DMA, pltpu.SemaphoreType.DMA,
        pltpu.SemaphoreType.DMA,
    ],
)
def _scatter2(lo_r, hi_r, p_r, sidx_r, lo_c, hi_c, p_c, sidx_c, z2, z1,
              accs, dens,
              sidx_res, p_res, v0, v1, v2, acc, den,
              l0, l1, l2, a0, a1, a2, dsem):
    cc = lax.axis_index("c")
    t = lax.axis_index("s")
    wid = t * 2 + cc
    vb = (v0, v1, v2)
    lsm = (l0, l1, l2)
    asx = (a0, a1, a2)
    rows = pl.ds(t * RPT, RPT)

    QW = D // 4

    def agg(srcarr, qc, with_den):
        for jj in range(RING - 1):
            pltpu.async_copy(
                srcarr.at[pl.ds(wid * EW + jj * CH, CH), pl.ds(qc * QW, QW)],
                vb[jj], lsm[jj])

        def step(j, first):
            sl = sidx_res.at[pl.ds(j * CH, CH)]

            def slot(b_):
                pltpu.make_async_copy(
                    srcarr.at[pl.ds(wid * EW, CH), pl.ds(0, QW)],
                    vb[b_], lsm[b_]).wait()
                pltpu.async_copy(vb[b_], acc.at[sl], asx[b_], add=True)
                if with_den:
                    pltpu.async_copy(p_res.at[pl.ds(j * CH, CH)], den.at[sl],
                                     dsem, add=True)

            def refill(b_):
                if not first:
                    pltpu.make_async_copy(
                        vb[b_], acc.at[sidx_res.at[pl.ds(0, CH)]],
                        asx[b_]).wait()
                pltpu.async_copy(
                    srcarr.at[pl.ds(wid * EW + (j + RING - 1) * CH, CH),
                              pl.ds(qc * QW, QW)],
                    vb[b_], lsm[b_])

            bp = lax.rem(j + RING - 1, RING)

            @pl.when(j + RING - 1 < NCH)
            def _():
                for b_ in range(RING):
                    pl.when(bp == b_)(functools.partial(refill, b_))

            b = lax.rem(j, RING)
            for b_ in range(RING):
                pl.when(b == b_)(functools.partial(slot, b_))

        step(0, True)
        lax.fori_loop(1, NCH, lambda j, c: (step(j, False), c)[1], 0,
                      unroll=False)
        for b_ in range(RING):
            pltpu.make_async_copy(vb[b_], acc.at[sidx_res.at[pl.ds(0, CH)]],
                                  asx[b_]).wait()
        if with_den:
            for _ in range(NCH):
                pltpu.make_async_copy(p_res.at[pl.ds(0, CH)],
                                      den.at[sidx_res.at[pl.ds(0, CH)]],
                                      dsem).wait()

    pltpu.sync_copy(z2, acc.at[rows])
    pltpu.sync_copy(z1, den.at[rows])
    plsc.subcore_barrier()

    for d, (lo, hi, p2, sidx) in enumerate(
            ((lo_r, hi_r, p_r, sidx_r), (lo_c, hi_c, p_c, sidx_c))):
        pltpu.sync_copy(sidx.at[wid], sidx_res)
        pltpu.sync_copy(p2.at[wid], p_res)
        for qi, (half, qc) in enumerate(((lo, 0), (lo, 1), (hi, 0), (hi, 1))):
            first = qi == 0
            agg(half, qc, first)
            plsc.subcore_barrier()
            pltpu.sync_copy(acc.at[rows], accs.at[d, cc, qi, rows])
            if first:
                pltpu.sync_copy(den.at[rows], dens.at[d, cc, rows])
                pltpu.sync_copy(z1, den.at[rows])
            pltpu.sync_copy(z2, acc.at[rows])
            plsc.subcore_barrier()


def _pad_idx(a, fill):
    ap = jnp.full((NW, EW), fill, jnp.int32)
    return ap.at[:, :EW_REAL].set(a.reshape(NW, EW_REAL).astype(jnp.int32))


def kernel(params, atoms, edge_index, edge_ids):
    x = params["atom_table"][atoms]
    src, dst = edge_index[0], edge_index[1]
    eid = edge_ids
    inv = 1.0 / math.sqrt(D)

    idxq_r = _pad_idx(eid * N + dst, 0)
    idxq_c = _pad_idx(eid * N + src, 0)
    idxkv_r = _pad_idx(src, 0)
    idxkv_c = _pad_idx(dst, 0)
    sidx_r = _pad_idx(dst, TRASH)
    sidx_c = _pad_idx(src, TRASH)
    z2 = jnp.zeros((RPT, D // 4), jnp.float32)
    z1 = jnp.zeros((RPT,), jnp.float32)

    for l in range(L):
        ws = jnp.stack([
            params["r2c_Wq"][l], params["r2c_Wk"][l], params["r2c_Wv"][l],
            params["c2r_Wq"][l], params["c2r_Wk"][l], params["c2r_Wv"][l],
        ])
        e3s = jnp.stack([
            jnp.tanh(params["edge_table"] @ params["r2c_We"][l]
                     + params["r2c_be"][l]) * inv,
            jnp.tanh(params["edge_table"] @ params["c2r_We"][l]
                     + params["c2r_be"][l]) * inv,
        ])
        q3r, kr, vr, q3c, kc, vc = _prep(x, ws, e3s)
        qgr, kgr, vgr, qgc, kgc, vgc = _gather6(
            q3r.reshape(3 * N, D), kr, vr, q3c.reshape(3 * N, D), kc, vc,
            idxq_r, idxkv_r, idxq_c, idxkv_c)
        lo_r, hi_r, p_r = _edge_math(qgr, kgr, vgr)
        lo_c, hi_c, p_c = _edge_math(qgc, kgc, vgc)
        accs, dens = _scatter2(lo_r, hi_r, p_r.reshape(NW, EW), sidx_r,
                               lo_c, hi_c, p_c.reshape(NW, EW), sidx_c,
                               z2, z1)
        outs = []
        for d in range(2):
            num = jnp.concatenate(
                [accs[d, 0, q] + accs[d, 1, q] for q in range(4)],
                axis=1)[:N]
            den = (dens[d, 0] + dens[d, 1])[:N]
            outs.append(num / (den[:, None] + 1e-16))
        h = jnp.concatenate(outs, axis=-1) @ params["ffn_W"][l]
        y = h + x
        mu = jnp.mean(y, axis=-1, keepdims=True)
        var = jnp.var(y, axis=-1, keepdims=True)
        x = (y - mu) / jnp.sqrt(var + 1e-5) * params["ln_g"][l] + params["ln_b"][l]
    return x
